# baseline (speedup 1.0000x reference)
import functools

import jax
import jax.numpy as jnp
from jax import lax
from jax.experimental import pallas as pl
from jax.experimental.pallas import tpu as pltpu
from jax.experimental.pallas import tpu_sc as plsc

NUM_EXPERTS = 8
TOP_K = 2
D_MODEL = 768
D_GATE_HID = 2 * D_MODEL
D_FF = 4 * D_MODEL
LB_COEF = 0.01

T_TILE = 256
R_TILE = 128
FF_TILE = 512
N_FF = D_FF // FF_TILE

NC, NS = 2, 16
NW = NC * NS


def _gate_kernel(x_ref, wg1_ref, bg1_ref, wg2_ref, bg2_ref,
                 i1_ref, i2_ref, w1_ref, w2_ref, usage_ref):
    x = x_ref[...].astype(jnp.bfloat16)
    h = jnp.maximum(jnp.dot(x, wg1_ref[...],
                            preferred_element_type=jnp.float32)
                    + bg1_ref[...], 0.0)
    logits = jnp.dot(h.astype(jnp.bfloat16), wg2_ref[...],
                     preferred_element_type=jnp.float32) + bg2_ref[...]
    m = jnp.max(logits, axis=-1, keepdims=True)
    e = jnp.exp(logits - m)
    scores = e / jnp.sum(e, axis=-1, keepdims=True)

    lane = jax.lax.broadcasted_iota(jnp.int32, scores.shape, 1)
    big = jnp.int32(NUM_EXPERTS)
    m1 = jnp.max(scores, axis=-1, keepdims=True)
    i1 = jnp.min(jnp.where(scores == m1, lane, big), axis=-1, keepdims=True)
    masked = jnp.where(lane == i1, -jnp.inf, scores)
    m2 = jnp.max(masked, axis=-1, keepdims=True)
    i2 = jnp.min(jnp.where(masked == m2, lane, big), axis=-1, keepdims=True)

    i1_ref[...] = i1
    i2_ref[...] = i2
    s = m1 + m2
    w1_ref[...] = m1 / s
    w2_ref[...] = m2 / s

    @pl.when(pl.program_id(0) == 0)
    def _init():
        usage_ref[...] = jnp.zeros_like(usage_ref)

    usage_ref[...] += jnp.sum(scores, axis=0, keepdims=True)


def _gmm_kernel(e_map_ref, x_ref, we1_ref, be1_ref, we2_ref, be2_ref,
                out_ref):
    x_bf = x_ref[...].astype(jnp.bfloat16)
    h = jnp.dot(x_bf, we1_ref[0],
                preferred_element_type=jnp.float32) + be1_ref[0]
    h = 0.5 * h * (1.0 + jax.lax.erf(h * 0.7071067811865476))
    out_ref[...] = jnp.dot(h.astype(jnp.bfloat16), we2_ref[0],
                           preferred_element_type=jnp.float32) + be2_ref[0]


def _make_dispatch(n_tok, p_rows):
    tpw = n_tok // NW
    mesh = plsc.VectorSubcoreMesh(core_axis_name="c", subcore_axis_name="s",
                                  num_cores=NC, num_subcores=NS)

    @functools.partial(
        pl.kernel,
        out_type=jax.ShapeDtypeStruct((p_rows, D_MODEL), jnp.float32),
        mesh=mesh,
        scratch_types=[
            pltpu.VMEM((tpw,), jnp.int32),
            pltpu.VMEM((tpw,), jnp.int32),
            pltpu.VMEM((tpw, D_MODEL), jnp.float32),
            pltpu.SemaphoreType.DMA,
        ],
    )
    def dispatch(x_hbm, idx0_hbm, idx1_hbm, out_hbm,
                 idx0_v, idx1_v, rows_v, sem):
        wid = lax.axis_index("s") * NC + lax.axis_index("c")
        base = wid * tpw
        c0 = pltpu.async_copy(idx0_hbm.at[pl.ds(base, tpw)], idx0_v, sem)
        c1 = pltpu.async_copy(idx1_hbm.at[pl.ds(base, tpw)], idx1_v, sem)
        c2 = pltpu.async_copy(x_hbm.at[pl.ds(base, tpw)], rows_v, sem)
        c0.wait()
        c1.wait()
        c2.wait()
        s0 = pltpu.async_copy(rows_v, out_hbm.at[idx0_v], sem)
        s1 = pltpu.async_copy(rows_v, out_hbm.at[idx1_v], sem)
        s0.wait()
        s1.wait()

    return dispatch


def _make_combine(n_tok):
    sub = 64
    n_sub = n_tok // (NW * sub)
    mesh = plsc.VectorSubcoreMesh(core_axis_name="c", subcore_axis_name="s",
                                  num_cores=NC, num_subcores=NS)

    @functools.partial(
        pl.kernel,
        out_type=jax.ShapeDtypeStruct((n_tok, D_MODEL), jnp.float32),
        mesh=mesh,
        scratch_types=[
            pltpu.VMEM((sub,), jnp.int32),
            pltpu.VMEM((sub,), jnp.int32),
            pltpu.VMEM((sub, 16), jnp.float32),
            pltpu.VMEM((sub, 16), jnp.float32),
            pltpu.VMEM((sub, D_MODEL), jnp.float32),
            pltpu.VMEM((sub, D_MODEL), jnp.float32),
            pltpu.SemaphoreType.DMA,
        ],
    )
    def combine(y_hbm, r0_hbm, r1_hbm, w0_hbm, w1_hbm, out_hbm,
                r0_v, r1_v, w0_v, w1_v, a_v, b_v, sem):
        wid = lax.axis_index("s") * NC + lax.axis_index("c")
        for s in range(n_sub):
            base = (wid * n_sub + s) * sub
            c0 = pltpu.async_copy(r0_hbm.at[pl.ds(base, sub)], r0_v, sem)
            c1 = pltpu.async_copy(r1_hbm.at[pl.ds(base, sub)], r1_v, sem)
            c2 = pltpu.async_copy(w0_hbm.at[pl.ds(base, sub)], w0_v, sem)
            c3 = pltpu.async_copy(w1_hbm.at[pl.ds(base, sub)], w1_v, sem)
            c0.wait()
            c1.wait()
            c2.wait()
            c3.wait()
            g0 = pltpu.async_copy(y_hbm.at[r0_v], a_v, sem)
            g1 = pltpu.async_copy(y_hbm.at[r1_v], b_v, sem)
            g0.wait()
            g1.wait()

            def body(i, _):
                wa = w0_v[i, :]
                wb = w1_v[i, :]
                for c in range(D_MODEL // 16):
                    a_v[i, pl.ds(c * 16, 16)] = (
                        wa * a_v[i, pl.ds(c * 16, 16)]
                        + wb * b_v[i, pl.ds(c * 16, 16)])
                return 0

            lax.fori_loop(0, sub, body, 0)
            pltpu.sync_copy(a_v, out_hbm.at[pl.ds(base, sub)])

    return combine


@jax.jit
def kernel(x, Wg1, bg1, Wg2, bg2, We1, be1, We2, be2):
    B, S, D = x.shape
    T = B * S
    x_flat = x.reshape(T, D)
    n_t = T // T_TILE

    i1, i2, w1, w2, usage_sum = pl.pallas_call(
        _gate_kernel,
        grid=(n_t,),
        in_specs=[
            pl.BlockSpec((T_TILE, D_MODEL), lambda t: (t, 0)),
            pl.BlockSpec((D_MODEL, D_GATE_HID), lambda t: (0, 0)),
            pl.BlockSpec((1, D_GATE_HID), lambda t: (0, 0)),
            pl.BlockSpec((D_GATE_HID, NUM_EXPERTS), lambda t: (0, 0)),
            pl.BlockSpec((1, NUM_EXPERTS), lambda t: (0, 0)),
        ],
        out_specs=(
            pl.BlockSpec((T_TILE, 1), lambda t: (t, 0)),
            pl.BlockSpec((T_TILE, 1), lambda t: (t, 0)),
            pl.BlockSpec((T_TILE, 1), lambda t: (t, 0)),
            pl.BlockSpec((T_TILE, 1), lambda t: (t, 0)),
            pl.BlockSpec((1, NUM_EXPERTS), lambda t: (0, 0)),
        ),
        out_shape=(
            jax.ShapeDtypeStruct((T, 1), jnp.int32),
            jax.ShapeDtypeStruct((T, 1), jnp.int32),
            jax.ShapeDtypeStruct((T, 1), jnp.float32),
            jax.ShapeDtypeStruct((T, 1), jnp.float32),
            jax.ShapeDtypeStruct((1, NUM_EXPERTS), jnp.float32),
        ),
    )(x_flat, Wg1.astype(jnp.bfloat16), bg1.reshape(1, -1),
      Wg2.astype(jnp.bfloat16), bg2.reshape(1, -1))

    e_flat = jnp.concatenate([i1, i2], axis=1).reshape(-1)
    oh = (e_flat[:, None] == jnp.arange(NUM_EXPERTS)[None, :]).astype(jnp.int32)
    cum = jnp.cumsum(oh, axis=0)
    rank = jnp.take_along_axis(cum, e_flat[:, None], axis=1)[:, 0] - 1
    counts = cum[-1]
    psize = ((counts + R_TILE - 1) // R_TILE) * R_TILE
    pstart = jnp.concatenate([jnp.zeros((1,), jnp.int32),
                              jnp.cumsum(psize)[:-1].astype(jnp.int32)])
    dest = pstart[e_flat] + rank
    dest2 = dest.reshape(T, TOP_K)
    idx0 = dest2[:, 0]
    idx1 = dest2[:, 1]

    P = T * TOP_K + NUM_EXPERTS * R_TILE
    n_tiles = P // R_TILE
    t_starts = jnp.arange(n_tiles, dtype=jnp.int32) * R_TILE
    e_of_t = (jnp.sum((pstart[None, :] <= t_starts[:, None]), axis=1) - 1
              ).astype(jnp.int32)

    x_sorted = _make_dispatch(T, P)(x_flat, idx0, idx1)

    y_sorted = pl.pallas_call(
        _gmm_kernel,
        grid_spec=pltpu.PrefetchScalarGridSpec(
            num_scalar_prefetch=1,
            grid=(n_tiles,),
            in_specs=[
                pl.BlockSpec((R_TILE, D_MODEL), lambda t, em: (t, 0)),
                pl.BlockSpec((1, D_MODEL, D_FF), lambda t, em: (em[t], 0, 0)),
                pl.BlockSpec((1, 1, D_FF), lambda t, em: (em[t], 0, 0)),
                pl.BlockSpec((1, D_FF, D_MODEL), lambda t, em: (em[t], 0, 0)),
                pl.BlockSpec((1, 1, D_MODEL), lambda t, em: (em[t], 0, 0)),
            ],
            out_specs=pl.BlockSpec((R_TILE, D_MODEL), lambda t, em: (t, 0)),
        ),
        out_shape=jax.ShapeDtypeStruct((P, D_MODEL), jnp.float32),
    )(e_of_t, x_sorted, We1.astype(jnp.bfloat16),
      be1.reshape(NUM_EXPERTS, 1, D_FF),
      We2.astype(jnp.bfloat16), be2.reshape(NUM_EXPERTS, 1, D_MODEL))

    w0b = jnp.broadcast_to(w1, (T, 16))
    w1b = jnp.broadcast_to(w2, (T, 16))
    out = _make_combine(T)(y_sorted, idx0, idx1, w0b, w1b)

    usage = usage_sum[0] / T
    ideal = 1.0 / NUM_EXPERTS
    lb_loss = LB_COEF * jnp.mean((usage - ideal) ** 2)
    return out.reshape(B, S, D), lb_loss


# speedup vs baseline: 1.2356x; 1.2356x over previous
import functools

import jax
import jax.numpy as jnp
from jax import lax
from jax.experimental import pallas as pl
from jax.experimental.pallas import tpu as pltpu
from jax.experimental.pallas import tpu_sc as plsc

NUM_EXPERTS = 8
TOP_K = 2
D_MODEL = 768
D_GATE_HID = 2 * D_MODEL
D_FF = 4 * D_MODEL
LB_COEF = 0.01

T_TILE = 256
R_TILE = 256
FF_TILE = 512
N_FF = D_FF // FF_TILE

NC, NS = 2, 16
NW = NC * NS


def _gate_kernel(x_ref, wg1_ref, bg1_ref, wg2_ref, bg2_ref,
                 i1_ref, i2_ref, w1_ref, w2_ref, usage_ref):
    x = x_ref[...].astype(jnp.bfloat16)
    h = jnp.maximum(jnp.dot(x, wg1_ref[...],
                            preferred_element_type=jnp.float32)
                    + bg1_ref[...], 0.0)
    logits = jnp.dot(h.astype(jnp.bfloat16), wg2_ref[...],
                     preferred_element_type=jnp.float32) + bg2_ref[...]
    m = jnp.max(logits, axis=-1, keepdims=True)
    e = jnp.exp(logits - m)
    scores = e / jnp.sum(e, axis=-1, keepdims=True)

    lane = jax.lax.broadcasted_iota(jnp.int32, scores.shape, 1)
    big = jnp.int32(NUM_EXPERTS)
    m1 = jnp.max(scores, axis=-1, keepdims=True)
    i1 = jnp.min(jnp.where(scores == m1, lane, big), axis=-1, keepdims=True)
    masked = jnp.where(lane == i1, -jnp.inf, scores)
    m2 = jnp.max(masked, axis=-1, keepdims=True)
    i2 = jnp.min(jnp.where(masked == m2, lane, big), axis=-1, keepdims=True)

    i1_ref[...] = i1
    i2_ref[...] = i2
    s = m1 + m2
    w1_ref[...] = m1 / s
    w2_ref[...] = m2 / s

    @pl.when(pl.program_id(0) == 0)
    def _init():
        usage_ref[...] = jnp.zeros_like(usage_ref)

    usage_ref[...] += jnp.sum(scores, axis=0, keepdims=True)


def _gmm_kernel(e_map_ref, x_ref, we1_ref, be1_ref, we2_ref, be2_ref,
                out_ref):
    h = jnp.dot(x_ref[...], we1_ref[0],
                preferred_element_type=jnp.float32,
                precision=jax.lax.Precision.DEFAULT) + be1_ref[0]
    h = 0.5 * h * (1.0 + jax.lax.erf(h * 0.7071067811865476))
    out_ref[...] = jnp.dot(h, we2_ref[0],
                           preferred_element_type=jnp.float32,
                           precision=jax.lax.Precision.DEFAULT) + be2_ref[0]


def _make_dispatch(n_tok, p_rows):
    tpw = n_tok // NW
    mesh = plsc.VectorSubcoreMesh(core_axis_name="c", subcore_axis_name="s",
                                  num_cores=NC, num_subcores=NS)

    @functools.partial(
        pl.kernel,
        out_type=jax.ShapeDtypeStruct((p_rows, D_MODEL), jnp.float32),
        mesh=mesh,
        scratch_types=[
            pltpu.VMEM((tpw,), jnp.int32),
            pltpu.VMEM((tpw,), jnp.int32),
            pltpu.VMEM((tpw, D_MODEL), jnp.float32),
            pltpu.SemaphoreType.DMA,
        ],
    )
    def dispatch(x_hbm, idx0_hbm, idx1_hbm, out_hbm,
                 idx0_v, idx1_v, rows_v, sem):
        wid = lax.axis_index("s") * NC + lax.axis_index("c")
        base = wid * tpw
        c0 = pltpu.async_copy(idx0_hbm.at[pl.ds(base, tpw)], idx0_v, sem)
        c1 = pltpu.async_copy(idx1_hbm.at[pl.ds(base, tpw)], idx1_v, sem)
        c2 = pltpu.async_copy(x_hbm.at[pl.ds(base, tpw)], rows_v, sem)
        c0.wait()
        c1.wait()
        c2.wait()
        s0 = pltpu.async_copy(rows_v, out_hbm.at[idx0_v], sem)
        s1 = pltpu.async_copy(rows_v, out_hbm.at[idx1_v], sem)
        s0.wait()
        s1.wait()

    return dispatch


def _make_combine(n_tok):
    sub = 64
    n_sub = n_tok // (NW * sub)
    mesh = plsc.VectorSubcoreMesh(core_axis_name="c", subcore_axis_name="s",
                                  num_cores=NC, num_subcores=NS)

    @functools.partial(
        pl.kernel,
        out_type=jax.ShapeDtypeStruct((n_tok, D_MODEL), jnp.float32),
        mesh=mesh,
        scratch_types=[
            pltpu.VMEM((sub,), jnp.int32),
            pltpu.VMEM((sub,), jnp.int32),
            pltpu.VMEM((sub, 16), jnp.float32),
            pltpu.VMEM((sub, 16), jnp.float32),
            pltpu.VMEM((sub, D_MODEL), jnp.float32),
            pltpu.VMEM((sub, D_MODEL), jnp.float32),
            pltpu.SemaphoreType.DMA,
        ],
    )
    def combine(y_hbm, r0_hbm, r1_hbm, w0_hbm, w1_hbm, out_hbm,
                r0_v, r1_v, w0_v, w1_v, a_v, b_v, sem):
        wid = lax.axis_index("s") * NC + lax.axis_index("c")
        for s in range(n_sub):
            base = (wid * n_sub + s) * sub
            c0 = pltpu.async_copy(r0_hbm.at[pl.ds(base, sub)], r0_v, sem)
            c1 = pltpu.async_copy(r1_hbm.at[pl.ds(base, sub)], r1_v, sem)
            c2 = pltpu.async_copy(w0_hbm.at[pl.ds(base, sub)], w0_v, sem)
            c3 = pltpu.async_copy(w1_hbm.at[pl.ds(base, sub)], w1_v, sem)
            c0.wait()
            c1.wait()
            c2.wait()
            c3.wait()
            g0 = pltpu.async_copy(y_hbm.at[r0_v], a_v, sem)
            g1 = pltpu.async_copy(y_hbm.at[r1_v], b_v, sem)
            g0.wait()
            g1.wait()

            def body(i, _):
                wa = w0_v[i, :]
                wb = w1_v[i, :]
                for c in range(D_MODEL // 16):
                    a_v[i, pl.ds(c * 16, 16)] = (
                        wa * a_v[i, pl.ds(c * 16, 16)]
                        + wb * b_v[i, pl.ds(c * 16, 16)])
                return 0

            lax.fori_loop(0, sub, body, 0)
            pltpu.sync_copy(a_v, out_hbm.at[pl.ds(base, sub)])

    return combine


@jax.jit
def kernel(x, Wg1, bg1, Wg2, bg2, We1, be1, We2, be2):
    B, S, D = x.shape
    T = B * S
    x_flat = x.reshape(T, D)
    n_t = T // T_TILE

    i1, i2, w1, w2, usage_sum = pl.pallas_call(
        _gate_kernel,
        grid=(n_t,),
        in_specs=[
            pl.BlockSpec((T_TILE, D_MODEL), lambda t: (t, 0)),
            pl.BlockSpec((D_MODEL, D_GATE_HID), lambda t: (0, 0)),
            pl.BlockSpec((1, D_GATE_HID), lambda t: (0, 0)),
            pl.BlockSpec((D_GATE_HID, NUM_EXPERTS), lambda t: (0, 0)),
            pl.BlockSpec((1, NUM_EXPERTS), lambda t: (0, 0)),
        ],
        out_specs=(
            pl.BlockSpec((T_TILE, 1), lambda t: (t, 0)),
            pl.BlockSpec((T_TILE, 1), lambda t: (t, 0)),
            pl.BlockSpec((T_TILE, 1), lambda t: (t, 0)),
            pl.BlockSpec((T_TILE, 1), lambda t: (t, 0)),
            pl.BlockSpec((1, NUM_EXPERTS), lambda t: (0, 0)),
        ),
        out_shape=(
            jax.ShapeDtypeStruct((T, 1), jnp.int32),
            jax.ShapeDtypeStruct((T, 1), jnp.int32),
            jax.ShapeDtypeStruct((T, 1), jnp.float32),
            jax.ShapeDtypeStruct((T, 1), jnp.float32),
            jax.ShapeDtypeStruct((1, NUM_EXPERTS), jnp.float32),
        ),
    )(x_flat, Wg1.astype(jnp.bfloat16), bg1.reshape(1, -1),
      Wg2.astype(jnp.bfloat16), bg2.reshape(1, -1))

    e_flat = jnp.concatenate([i1, i2], axis=1).reshape(-1)
    oh = (e_flat[:, None] == jnp.arange(NUM_EXPERTS)[None, :]).astype(jnp.int32)
    cum = jnp.cumsum(oh, axis=0)
    rank = jnp.take_along_axis(cum, e_flat[:, None], axis=1)[:, 0] - 1
    counts = cum[-1]
    psize = ((counts + R_TILE - 1) // R_TILE) * R_TILE
    pstart = jnp.concatenate([jnp.zeros((1,), jnp.int32),
                              jnp.cumsum(psize)[:-1].astype(jnp.int32)])
    dest = pstart[e_flat] + rank
    dest2 = dest.reshape(T, TOP_K)
    idx0 = dest2[:, 0]
    idx1 = dest2[:, 1]

    P = T * TOP_K + NUM_EXPERTS * R_TILE
    n_tiles = P // R_TILE
    t_starts = jnp.arange(n_tiles, dtype=jnp.int32) * R_TILE
    e_of_t = (jnp.sum((pstart[None, :] <= t_starts[:, None]), axis=1) - 1
              ).astype(jnp.int32)

    x_sorted = _make_dispatch(T, P)(x_flat, idx0, idx1)

    y_sorted = pl.pallas_call(
        _gmm_kernel,
        grid_spec=pltpu.PrefetchScalarGridSpec(
            num_scalar_prefetch=1,
            grid=(n_tiles,),
            in_specs=[
                pl.BlockSpec((R_TILE, D_MODEL), lambda t, em: (t, 0)),
                pl.BlockSpec((1, D_MODEL, D_FF), lambda t, em: (em[t], 0, 0)),
                pl.BlockSpec((1, 1, D_FF), lambda t, em: (em[t], 0, 0)),
                pl.BlockSpec((1, D_FF, D_MODEL), lambda t, em: (em[t], 0, 0)),
                pl.BlockSpec((1, 1, D_MODEL), lambda t, em: (em[t], 0, 0)),
            ],
            out_specs=pl.BlockSpec((R_TILE, D_MODEL), lambda t, em: (t, 0)),
        ),
        out_shape=jax.ShapeDtypeStruct((P, D_MODEL), jnp.float32),
    )(e_of_t, x_sorted, We1,
      be1.reshape(NUM_EXPERTS, 1, D_FF),
      We2, be2.reshape(NUM_EXPERTS, 1, D_MODEL))

    w0b = jnp.broadcast_to(w1, (T, 16))
    w1b = jnp.broadcast_to(w2, (T, 16))
    out = _make_combine(T)(y_sorted, idx0, idx1, w0b, w1b)

    usage = usage_sum[0] / T
    ideal = 1.0 / NUM_EXPERTS
    lb_loss = LB_COEF * jnp.mean((usage - ideal) ** 2)
    return out.reshape(B, S, D), lb_loss


# speedup vs baseline: 1.2899x; 1.0440x over previous
import functools

import jax
import jax.numpy as jnp
from jax import lax
from jax.experimental import pallas as pl
from jax.experimental.pallas import tpu as pltpu
from jax.experimental.pallas import tpu_sc as plsc

NUM_EXPERTS = 8
TOP_K = 2
D_MODEL = 768
D_GATE_HID = 2 * D_MODEL
D_FF = 4 * D_MODEL
LB_COEF = 0.01

T_TILE = 256
R_TILE = 256
FF_TILE = 512
N_FF = D_FF // FF_TILE

NC, NS = 2, 16
NW = NC * NS


def _gate_kernel(x_ref, wg1_ref, bg1_ref, wg2_ref, bg2_ref,
                 i1_ref, i2_ref, w1_ref, w2_ref, usage_ref,
                 rank1_ref, rank2_ref, counts_ref):
    x = x_ref[...].astype(jnp.bfloat16)
    h = jnp.maximum(jnp.dot(x, wg1_ref[...],
                            preferred_element_type=jnp.float32)
                    + bg1_ref[...], 0.0)
    logits = jnp.dot(h.astype(jnp.bfloat16), wg2_ref[...],
                     preferred_element_type=jnp.float32) + bg2_ref[...]
    m = jnp.max(logits, axis=-1, keepdims=True)
    e = jnp.exp(logits - m)
    scores = e / jnp.sum(e, axis=-1, keepdims=True)

    lane = jax.lax.broadcasted_iota(jnp.int32, scores.shape, 1)
    big = jnp.int32(NUM_EXPERTS)
    m1 = jnp.max(scores, axis=-1, keepdims=True)
    i1 = jnp.min(jnp.where(scores == m1, lane, big), axis=-1, keepdims=True)
    masked = jnp.where(lane == i1, -jnp.inf, scores)
    m2 = jnp.max(masked, axis=-1, keepdims=True)
    i2 = jnp.min(jnp.where(masked == m2, lane, big), axis=-1, keepdims=True)

    i1_ref[...] = i1
    i2_ref[...] = i2
    s = m1 + m2
    w1_ref[...] = m1 / s
    w2_ref[...] = m2 / s

    @pl.when(pl.program_id(0) == 0)
    def _init():
        usage_ref[...] = jnp.zeros_like(usage_ref)
        counts_ref[...] = jnp.zeros_like(counts_ref)

    usage_ref[...] += jnp.sum(scores, axis=0, keepdims=True)

    oh1 = (lane == i1).astype(jnp.float32)
    oh2 = (lane == i2).astype(jnp.float32)
    ohsum = oh1 + oh2
    r_iota = jax.lax.broadcasted_iota(jnp.int32, (T_TILE, T_TILE), 0)
    c_iota = jax.lax.broadcasted_iota(jnp.int32, (T_TILE, T_TILE), 1)
    tri = (r_iota >= c_iota).astype(jnp.float32)
    cum = jnp.dot(tri, ohsum, preferred_element_type=jnp.float32)
    carry = counts_ref[...].astype(jnp.float32) + cum - ohsum
    rank1_ref[...] = jnp.sum(carry * oh1, axis=1,
                             keepdims=True).astype(jnp.int32)
    rank2_ref[...] = jnp.sum(carry * oh2, axis=1,
                             keepdims=True).astype(jnp.int32)
    counts_ref[...] += jnp.sum(ohsum, axis=0,
                               keepdims=True).astype(jnp.int32)


def _gmm_kernel(e_map_ref, x_ref, we1_ref, be1_ref, we2_ref, be2_ref,
                out_ref):
    h = jnp.dot(x_ref[...], we1_ref[0],
                preferred_element_type=jnp.float32,
                precision=jax.lax.Precision.DEFAULT) + be1_ref[0]
    h = 0.5 * h * (1.0 + jax.lax.erf(h * 0.7071067811865476))
    out_ref[...] = jnp.dot(h, we2_ref[0],
                           preferred_element_type=jnp.float32,
                           precision=jax.lax.Precision.DEFAULT) + be2_ref[0]


def _make_dispatch(n_tok, p_rows):
    tpw = n_tok // NW
    mesh = plsc.VectorSubcoreMesh(core_axis_name="c", subcore_axis_name="s",
                                  num_cores=NC, num_subcores=NS)

    @functools.partial(
        pl.kernel,
        out_type=jax.ShapeDtypeStruct((p_rows, D_MODEL), jnp.float32),
        mesh=mesh,
        scratch_types=[
            pltpu.VMEM((tpw,), jnp.int32),
            pltpu.VMEM((tpw,), jnp.int32),
            pltpu.VMEM((tpw, D_MODEL), jnp.float32),
            pltpu.SemaphoreType.DMA,
        ],
    )
    def dispatch(x_hbm, idx0_hbm, idx1_hbm, out_hbm,
                 idx0_v, idx1_v, rows_v, sem):
        wid = lax.axis_index("s") * NC + lax.axis_index("c")
        base = wid * tpw
        c0 = pltpu.async_copy(idx0_hbm.at[pl.ds(base, tpw)], idx0_v, sem)
        c1 = pltpu.async_copy(idx1_hbm.at[pl.ds(base, tpw)], idx1_v, sem)
        c2 = pltpu.async_copy(x_hbm.at[pl.ds(base, tpw)], rows_v, sem)
        c0.wait()
        c1.wait()
        c2.wait()
        s0 = pltpu.async_copy(rows_v, out_hbm.at[idx0_v], sem)
        s1 = pltpu.async_copy(rows_v, out_hbm.at[idx1_v], sem)
        s0.wait()
        s1.wait()

    return dispatch


def _make_combine(n_tok):
    sub = 64
    n_sub = n_tok // (NW * sub)
    mesh = plsc.VectorSubcoreMesh(core_axis_name="c", subcore_axis_name="s",
                                  num_cores=NC, num_subcores=NS)

    @functools.partial(
        pl.kernel,
        out_type=jax.ShapeDtypeStruct((n_tok, D_MODEL), jnp.float32),
        mesh=mesh,
        scratch_types=[
            pltpu.VMEM((sub,), jnp.int32),
            pltpu.VMEM((sub,), jnp.int32),
            pltpu.VMEM((sub, 16), jnp.float32),
            pltpu.VMEM((sub, 16), jnp.float32),
            pltpu.VMEM((sub, D_MODEL), jnp.float32),
            pltpu.VMEM((sub, D_MODEL), jnp.float32),
            pltpu.SemaphoreType.DMA,
        ],
    )
    def combine(y_hbm, r0_hbm, r1_hbm, w0_hbm, w1_hbm, out_hbm,
                r0_v, r1_v, w0_v, w1_v, a_v, b_v, sem):
        wid = lax.axis_index("s") * NC + lax.axis_index("c")
        for s in range(n_sub):
            base = (wid * n_sub + s) * sub
            c0 = pltpu.async_copy(r0_hbm.at[pl.ds(base, sub)], r0_v, sem)
            c1 = pltpu.async_copy(r1_hbm.at[pl.ds(base, sub)], r1_v, sem)
            c2 = pltpu.async_copy(w0_hbm.at[pl.ds(base, sub)], w0_v, sem)
            c3 = pltpu.async_copy(w1_hbm.at[pl.ds(base, sub)], w1_v, sem)
            c0.wait()
            c1.wait()
            c2.wait()
            c3.wait()
            g0 = pltpu.async_copy(y_hbm.at[r0_v], a_v, sem)
            g1 = pltpu.async_copy(y_hbm.at[r1_v], b_v, sem)
            g0.wait()
            g1.wait()

            def body(i, _):
                wa = w0_v[i, :]
                wb = w1_v[i, :]
                for c in range(D_MODEL // 16):
                    a_v[i, pl.ds(c * 16, 16)] = (
                        wa * a_v[i, pl.ds(c * 16, 16)]
                        + wb * b_v[i, pl.ds(c * 16, 16)])
                return 0

            lax.fori_loop(0, sub, body, 0)
            pltpu.sync_copy(a_v, out_hbm.at[pl.ds(base, sub)])

    return combine


@jax.jit
def kernel(x, Wg1, bg1, Wg2, bg2, We1, be1, We2, be2):
    B, S, D = x.shape
    T = B * S
    x_flat = x.reshape(T, D)
    n_t = T // T_TILE

    i1, i2, w1, w2, usage_sum, rank1, rank2, counts_out = pl.pallas_call(
        _gate_kernel,
        grid=(n_t,),
        in_specs=[
            pl.BlockSpec((T_TILE, D_MODEL), lambda t: (t, 0)),
            pl.BlockSpec((D_MODEL, D_GATE_HID), lambda t: (0, 0)),
            pl.BlockSpec((1, D_GATE_HID), lambda t: (0, 0)),
            pl.BlockSpec((D_GATE_HID, NUM_EXPERTS), lambda t: (0, 0)),
            pl.BlockSpec((1, NUM_EXPERTS), lambda t: (0, 0)),
        ],
        out_specs=(
            pl.BlockSpec((T_TILE, 1), lambda t: (t, 0)),
            pl.BlockSpec((T_TILE, 1), lambda t: (t, 0)),
            pl.BlockSpec((T_TILE, 1), lambda t: (t, 0)),
            pl.BlockSpec((T_TILE, 1), lambda t: (t, 0)),
            pl.BlockSpec((1, NUM_EXPERTS), lambda t: (0, 0)),
            pl.BlockSpec((T_TILE, 1), lambda t: (t, 0)),
            pl.BlockSpec((T_TILE, 1), lambda t: (t, 0)),
            pl.BlockSpec((1, NUM_EXPERTS), lambda t: (0, 0)),
        ),
        out_shape=(
            jax.ShapeDtypeStruct((T, 1), jnp.int32),
            jax.ShapeDtypeStruct((T, 1), jnp.int32),
            jax.ShapeDtypeStruct((T, 1), jnp.float32),
            jax.ShapeDtypeStruct((T, 1), jnp.float32),
            jax.ShapeDtypeStruct((1, NUM_EXPERTS), jnp.float32),
            jax.ShapeDtypeStruct((T, 1), jnp.int32),
            jax.ShapeDtypeStruct((T, 1), jnp.int32),
            jax.ShapeDtypeStruct((1, NUM_EXPERTS), jnp.int32),
        ),
    )(x_flat, Wg1.astype(jnp.bfloat16), bg1.reshape(1, -1),
      Wg2.astype(jnp.bfloat16), bg2.reshape(1, -1))

    counts = counts_out[0]
    psize = ((counts + R_TILE - 1) // R_TILE) * R_TILE
    pstart = jnp.concatenate([jnp.zeros((1,), jnp.int32),
                              jnp.cumsum(psize)[:-1].astype(jnp.int32)])
    idx0 = pstart[i1[:, 0]] + rank1[:, 0]
    idx1 = pstart[i2[:, 0]] + rank2[:, 0]

    P = T * TOP_K + NUM_EXPERTS * R_TILE
    n_tiles = P // R_TILE
    t_starts = jnp.arange(n_tiles, dtype=jnp.int32) * R_TILE
    e_of_t = (jnp.sum((pstart[None, :] <= t_starts[:, None]), axis=1) - 1
              ).astype(jnp.int32)

    x_sorted = _make_dispatch(T, P)(x_flat, idx0, idx1)

    y_sorted = pl.pallas_call(
        _gmm_kernel,
        grid_spec=pltpu.PrefetchScalarGridSpec(
            num_scalar_prefetch=1,
            grid=(n_tiles,),
            in_specs=[
                pl.BlockSpec((R_TILE, D_MODEL), lambda t, em: (t, 0)),
                pl.BlockSpec((1, D_MODEL, D_FF), lambda t, em: (em[t], 0, 0)),
                pl.BlockSpec((1, 1, D_FF), lambda t, em: (em[t], 0, 0)),
                pl.BlockSpec((1, D_FF, D_MODEL), lambda t, em: (em[t], 0, 0)),
                pl.BlockSpec((1, 1, D_MODEL), lambda t, em: (em[t], 0, 0)),
            ],
            out_specs=pl.BlockSpec((R_TILE, D_MODEL), lambda t, em: (t, 0)),
        ),
        out_shape=jax.ShapeDtypeStruct((P, D_MODEL), jnp.float32),
    )(e_of_t, x_sorted, We1,
      be1.reshape(NUM_EXPERTS, 1, D_FF),
      We2, be2.reshape(NUM_EXPERTS, 1, D_MODEL))

    w0b = jnp.broadcast_to(w1, (T, 16))
    w1b = jnp.broadcast_to(w2, (T, 16))
    out = _make_combine(T)(y_sorted, idx0, idx1, w0b, w1b)

    usage = usage_sum[0] / T
    ideal = 1.0 / NUM_EXPERTS
    lb_loss = LB_COEF * jnp.mean((usage - ideal) ** 2)
    return out.reshape(B, S, D), lb_loss


# speedup vs baseline: 1.3451x; 1.0428x over previous
import functools

import jax
import jax.numpy as jnp
from jax import lax
from jax.experimental import pallas as pl
from jax.experimental.pallas import tpu as pltpu
from jax.experimental.pallas import tpu_sc as plsc

NUM_EXPERTS = 8
TOP_K = 2
D_MODEL = 768
D_GATE_HID = 2 * D_MODEL
D_FF = 4 * D_MODEL
LB_COEF = 0.01

T_TILE = 256
R_TILE = 256
FF_TILE = 512
N_FF = D_FF // FF_TILE

NC, NS = 2, 16
NW = NC * NS


def _gate_kernel(x_ref, wg1_ref, bg1_ref, wg2_ref, bg2_ref,
                 i1_ref, i2_ref, w1_ref, w2_ref, usage_ref,
                 rank1_ref, rank2_ref, counts_ref):
    x = x_ref[...].astype(jnp.bfloat16)
    h = jnp.maximum(jnp.dot(x, wg1_ref[...],
                            preferred_element_type=jnp.float32)
                    + bg1_ref[...], 0.0)
    logits = jnp.dot(h.astype(jnp.bfloat16), wg2_ref[...],
                     preferred_element_type=jnp.float32) + bg2_ref[...]
    m = jnp.max(logits, axis=-1, keepdims=True)
    e = jnp.exp(logits - m)
    scores = e / jnp.sum(e, axis=-1, keepdims=True)

    lane = jax.lax.broadcasted_iota(jnp.int32, scores.shape, 1)
    big = jnp.int32(NUM_EXPERTS)
    m1 = jnp.max(scores, axis=-1, keepdims=True)
    i1 = jnp.min(jnp.where(scores == m1, lane, big), axis=-1, keepdims=True)
    masked = jnp.where(lane == i1, -jnp.inf, scores)
    m2 = jnp.max(masked, axis=-1, keepdims=True)
    i2 = jnp.min(jnp.where(masked == m2, lane, big), axis=-1, keepdims=True)

    i1_ref[...] = i1
    i2_ref[...] = i2
    s = m1 + m2
    w1_ref[...] = m1 / s
    w2_ref[...] = m2 / s

    @pl.when(pl.program_id(0) == 0)
    def _init():
        usage_ref[...] = jnp.zeros_like(usage_ref)
        counts_ref[...] = jnp.zeros_like(counts_ref)

    usage_ref[...] += jnp.sum(scores, axis=0, keepdims=True)

    oh1 = (lane == i1).astype(jnp.float32)
    oh2 = (lane == i2).astype(jnp.float32)
    ohsum = oh1 + oh2
    r_iota = jax.lax.broadcasted_iota(jnp.int32, (T_TILE, T_TILE), 0)
    c_iota = jax.lax.broadcasted_iota(jnp.int32, (T_TILE, T_TILE), 1)
    tri = (r_iota >= c_iota).astype(jnp.float32)
    cum = jnp.dot(tri, ohsum, preferred_element_type=jnp.float32)
    carry = counts_ref[...].astype(jnp.float32) + cum - ohsum
    rank1_ref[...] = jnp.sum(carry * oh1, axis=1,
                             keepdims=True).astype(jnp.int32)
    rank2_ref[...] = jnp.sum(carry * oh2, axis=1,
                             keepdims=True).astype(jnp.int32)
    counts_ref[...] += jnp.sum(ohsum, axis=0,
                               keepdims=True).astype(jnp.int32)


def _gmm_kernel(e_map_ref, na_ref, x_ref, we1_ref, be1_ref, we2_ref,
                be2_ref, out_ref):
    @pl.when(pl.program_id(0) < na_ref[0])
    def _active():
        h = jnp.dot(x_ref[...], we1_ref[0],
                    preferred_element_type=jnp.float32,
                    precision=jax.lax.Precision.DEFAULT) + be1_ref[0]
        h = 0.5 * h * (1.0 + jax.lax.erf(h * 0.7071067811865476))
        out_ref[...] = jnp.dot(h, we2_ref[0],
                               preferred_element_type=jnp.float32,
                               precision=jax.lax.Precision.DEFAULT) + be2_ref[0]


def _make_dispatch(n_tok, p_rows):
    tpw = n_tok // NW
    mesh = plsc.VectorSubcoreMesh(core_axis_name="c", subcore_axis_name="s",
                                  num_cores=NC, num_subcores=NS)

    @functools.partial(
        pl.kernel,
        out_type=jax.ShapeDtypeStruct((p_rows, D_MODEL), jnp.float32),
        mesh=mesh,
        scratch_types=[
            pltpu.VMEM((tpw,), jnp.int32),
            pltpu.VMEM((tpw,), jnp.int32),
            pltpu.VMEM((tpw, D_MODEL), jnp.float32),
            pltpu.SemaphoreType.DMA,
        ],
    )
    def dispatch(x_hbm, idx0_hbm, idx1_hbm, out_hbm,
                 idx0_v, idx1_v, rows_v, sem):
        wid = lax.axis_index("s") * NC + lax.axis_index("c")
        base = wid * tpw
        c0 = pltpu.async_copy(idx0_hbm.at[pl.ds(base, tpw)], idx0_v, sem)
        c1 = pltpu.async_copy(idx1_hbm.at[pl.ds(base, tpw)], idx1_v, sem)
        c2 = pltpu.async_copy(x_hbm.at[pl.ds(base, tpw)], rows_v, sem)
        c0.wait()
        c1.wait()
        c2.wait()
        s0 = pltpu.async_copy(rows_v, out_hbm.at[idx0_v], sem)
        s1 = pltpu.async_copy(rows_v, out_hbm.at[idx1_v], sem)
        s0.wait()
        s1.wait()

    return dispatch


def _make_combine(n_tok):
    sub = 64
    n_sub = n_tok // (NW * sub)
    mesh = plsc.VectorSubcoreMesh(core_axis_name="c", subcore_axis_name="s",
                                  num_cores=NC, num_subcores=NS)

    @functools.partial(
        pl.kernel,
        out_type=jax.ShapeDtypeStruct((n_tok, D_MODEL), jnp.float32),
        mesh=mesh,
        scratch_types=[
            pltpu.VMEM((sub,), jnp.int32),
            pltpu.VMEM((sub,), jnp.int32),
            pltpu.VMEM((sub, 16), jnp.float32),
            pltpu.VMEM((sub, 16), jnp.float32),
            pltpu.VMEM((sub, D_MODEL), jnp.float32),
            pltpu.VMEM((sub, D_MODEL), jnp.float32),
            pltpu.SemaphoreType.DMA,
        ],
    )
    def combine(y_hbm, r0_hbm, r1_hbm, w0_hbm, w1_hbm, out_hbm,
                r0_v, r1_v, w0_v, w1_v, a_v, b_v, sem):
        wid = lax.axis_index("s") * NC + lax.axis_index("c")
        for s in range(n_sub):
            base = (wid * n_sub + s) * sub
            c0 = pltpu.async_copy(r0_hbm.at[pl.ds(base, sub)], r0_v, sem)
            c1 = pltpu.async_copy(r1_hbm.at[pl.ds(base, sub)], r1_v, sem)
            c2 = pltpu.async_copy(w0_hbm.at[pl.ds(base, sub)], w0_v, sem)
            c3 = pltpu.async_copy(w1_hbm.at[pl.ds(base, sub)], w1_v, sem)
            c0.wait()
            c1.wait()
            c2.wait()
            c3.wait()
            g0 = pltpu.async_copy(y_hbm.at[r0_v], a_v, sem)
            g1 = pltpu.async_copy(y_hbm.at[r1_v], b_v, sem)
            g0.wait()
            g1.wait()

            def body(i, _):
                wa = w0_v[i, :]
                wb = w1_v[i, :]
                for c in range(D_MODEL // 16):
                    a_v[i, pl.ds(c * 16, 16)] = (
                        wa * a_v[i, pl.ds(c * 16, 16)]
                        + wb * b_v[i, pl.ds(c * 16, 16)])
                return 0

            lax.fori_loop(0, sub, body, 0)
            pltpu.sync_copy(a_v, out_hbm.at[pl.ds(base, sub)])

    return combine


@jax.jit
def kernel(x, Wg1, bg1, Wg2, bg2, We1, be1, We2, be2):
    B, S, D = x.shape
    T = B * S
    x_flat = x.reshape(T, D)
    n_t = T // T_TILE

    i1, i2, w1, w2, usage_sum, rank1, rank2, counts_out = pl.pallas_call(
        _gate_kernel,
        grid=(n_t,),
        in_specs=[
            pl.BlockSpec((T_TILE, D_MODEL), lambda t: (t, 0)),
            pl.BlockSpec((D_MODEL, D_GATE_HID), lambda t: (0, 0)),
            pl.BlockSpec((1, D_GATE_HID), lambda t: (0, 0)),
            pl.BlockSpec((D_GATE_HID, NUM_EXPERTS), lambda t: (0, 0)),
            pl.BlockSpec((1, NUM_EXPERTS), lambda t: (0, 0)),
        ],
        out_specs=(
            pl.BlockSpec((T_TILE, 1), lambda t: (t, 0)),
            pl.BlockSpec((T_TILE, 1), lambda t: (t, 0)),
            pl.BlockSpec((T_TILE, 1), lambda t: (t, 0)),
            pl.BlockSpec((T_TILE, 1), lambda t: (t, 0)),
            pl.BlockSpec((1, NUM_EXPERTS), lambda t: (0, 0)),
            pl.BlockSpec((T_TILE, 1), lambda t: (t, 0)),
            pl.BlockSpec((T_TILE, 1), lambda t: (t, 0)),
            pl.BlockSpec((1, NUM_EXPERTS), lambda t: (0, 0)),
        ),
        out_shape=(
            jax.ShapeDtypeStruct((T, 1), jnp.int32),
            jax.ShapeDtypeStruct((T, 1), jnp.int32),
            jax.ShapeDtypeStruct((T, 1), jnp.float32),
            jax.ShapeDtypeStruct((T, 1), jnp.float32),
            jax.ShapeDtypeStruct((1, NUM_EXPERTS), jnp.float32),
            jax.ShapeDtypeStruct((T, 1), jnp.int32),
            jax.ShapeDtypeStruct((T, 1), jnp.int32),
            jax.ShapeDtypeStruct((1, NUM_EXPERTS), jnp.int32),
        ),
    )(x_flat, Wg1.astype(jnp.bfloat16), bg1.reshape(1, -1),
      Wg2.astype(jnp.bfloat16), bg2.reshape(1, -1))

    counts = counts_out[0]
    psize = ((counts + R_TILE - 1) // R_TILE) * R_TILE
    pstart = jnp.concatenate([jnp.zeros((1,), jnp.int32),
                              jnp.cumsum(psize)[:-1].astype(jnp.int32)])
    idx0 = pstart[i1[:, 0]] + rank1[:, 0]
    idx1 = pstart[i2[:, 0]] + rank2[:, 0]

    P = T * TOP_K + NUM_EXPERTS * R_TILE
    n_tiles = P // R_TILE
    t_starts = jnp.arange(n_tiles, dtype=jnp.int32) * R_TILE
    e_of_t = (jnp.sum((pstart[None, :] <= t_starts[:, None]), axis=1) - 1
              ).astype(jnp.int32)

    x_sorted = _make_dispatch(T, P)(x_flat, idx0, idx1)

    n_active = (pstart[-1] + psize[-1] + R_TILE - 1) // R_TILE
    y_sorted = pl.pallas_call(
        _gmm_kernel,
        grid_spec=pltpu.PrefetchScalarGridSpec(
            num_scalar_prefetch=2,
            grid=(n_tiles,),
            in_specs=[
                pl.BlockSpec((R_TILE, D_MODEL), lambda t, em, na: (t, 0)),
                pl.BlockSpec((1, D_MODEL, D_FF),
                             lambda t, em, na: (em[t], 0, 0)),
                pl.BlockSpec((1, 1, D_FF), lambda t, em, na: (em[t], 0, 0)),
                pl.BlockSpec((1, D_FF, D_MODEL),
                             lambda t, em, na: (em[t], 0, 0)),
                pl.BlockSpec((1, 1, D_MODEL),
                             lambda t, em, na: (em[t], 0, 0)),
            ],
            out_specs=pl.BlockSpec((R_TILE, D_MODEL),
                                   lambda t, em, na: (t, 0)),
        ),
        out_shape=jax.ShapeDtypeStruct((P, D_MODEL), jnp.float32),
    )(e_of_t, n_active.reshape(1), x_sorted, We1,
      be1.reshape(NUM_EXPERTS, 1, D_FF),
      We2, be2.reshape(NUM_EXPERTS, 1, D_MODEL))

    w0b = jnp.broadcast_to(w1, (T, 16))
    w1b = jnp.broadcast_to(w2, (T, 16))
    out = _make_combine(T)(y_sorted, idx0, idx1, w0b, w1b)

    usage = usage_sum[0] / T
    ideal = 1.0 / NUM_EXPERTS
    lb_loss = LB_COEF * jnp.mean((usage - ideal) ** 2)
    return out.reshape(B, S, D), lb_loss


# speedup vs baseline: 1.3541x; 1.0067x over previous
import functools

import jax
import jax.numpy as jnp
from jax import lax
from jax.experimental import pallas as pl
from jax.experimental.pallas import tpu as pltpu
from jax.experimental.pallas import tpu_sc as plsc

NUM_EXPERTS = 8
TOP_K = 2
D_MODEL = 768
D_GATE_HID = 2 * D_MODEL
D_FF = 4 * D_MODEL
LB_COEF = 0.01

T_TILE = 256
R_TILE = 256
FF_TILE = 512
N_FF = D_FF // FF_TILE

NC, NS = 2, 16
NW = NC * NS


def _gate_kernel(x_ref, wg1_ref, bg1_ref, wg2_ref, bg2_ref,
                 i1_ref, i2_ref, w1_ref, w2_ref, usage_ref,
                 rank1_ref, rank2_ref, counts_ref):
    x = x_ref[...].astype(jnp.bfloat16)
    h = jnp.maximum(jnp.dot(x, wg1_ref[...],
                            preferred_element_type=jnp.float32)
                    + bg1_ref[...], 0.0)
    logits = jnp.dot(h.astype(jnp.bfloat16), wg2_ref[...],
                     preferred_element_type=jnp.float32) + bg2_ref[...]
    m = jnp.max(logits, axis=-1, keepdims=True)
    e = jnp.exp(logits - m)
    scores = e / jnp.sum(e, axis=-1, keepdims=True)

    lane = jax.lax.broadcasted_iota(jnp.int32, scores.shape, 1)
    big = jnp.int32(NUM_EXPERTS)
    m1 = jnp.max(scores, axis=-1, keepdims=True)
    i1 = jnp.min(jnp.where(scores == m1, lane, big), axis=-1, keepdims=True)
    masked = jnp.where(lane == i1, -jnp.inf, scores)
    m2 = jnp.max(masked, axis=-1, keepdims=True)
    i2 = jnp.min(jnp.where(masked == m2, lane, big), axis=-1, keepdims=True)

    i1_ref[...] = i1
    i2_ref[...] = i2
    s = m1 + m2
    w1_ref[...] = m1 / s
    w2_ref[...] = m2 / s

    @pl.when(pl.program_id(0) == 0)
    def _init():
        usage_ref[...] = jnp.zeros_like(usage_ref)
        counts_ref[...] = jnp.zeros_like(counts_ref)

    usage_ref[...] += jnp.sum(scores, axis=0, keepdims=True)

    oh1 = (lane == i1).astype(jnp.float32)
    oh2 = (lane == i2).astype(jnp.float32)
    ohsum = oh1 + oh2
    r_iota = jax.lax.broadcasted_iota(jnp.int32, (T_TILE, T_TILE), 0)
    c_iota = jax.lax.broadcasted_iota(jnp.int32, (T_TILE, T_TILE), 1)
    tri = (r_iota >= c_iota).astype(jnp.float32)
    cum = jnp.dot(tri, ohsum, preferred_element_type=jnp.float32)
    carry = counts_ref[...].astype(jnp.float32) + cum - ohsum
    rank1_ref[...] = jnp.sum(carry * oh1, axis=1,
                             keepdims=True).astype(jnp.int32)
    rank2_ref[...] = jnp.sum(carry * oh2, axis=1,
                             keepdims=True).astype(jnp.int32)
    counts_ref[...] += jnp.sum(ohsum, axis=0,
                               keepdims=True).astype(jnp.int32)


def _gmm_kernel(e_map_ref, na_ref, x_ref, we1_ref, be1_ref, we2_ref,
                be2_ref, out_ref):
    @pl.when(pl.program_id(0) < na_ref[0])
    def _active():
        h = jnp.dot(x_ref[...], we1_ref[0],
                    preferred_element_type=jnp.float32,
                    precision=jax.lax.Precision.DEFAULT) + be1_ref[0]
        h = 0.5 * h * (1.0 + jax.lax.erf(h * 0.7071067811865476))
        out_ref[...] = jnp.dot(h, we2_ref[0],
                               preferred_element_type=jnp.float32,
                               precision=jax.lax.Precision.DEFAULT) + be2_ref[0]


def _make_dispatch(n_tok, p_rows):
    tpw = n_tok // NW
    mesh = plsc.VectorSubcoreMesh(core_axis_name="c", subcore_axis_name="s",
                                  num_cores=NC, num_subcores=NS)

    @functools.partial(
        pl.kernel,
        out_type=jax.ShapeDtypeStruct((p_rows, D_MODEL), jnp.float32),
        mesh=mesh,
        scratch_types=[
            pltpu.VMEM((tpw,), jnp.int32),
            pltpu.VMEM((tpw,), jnp.int32),
            pltpu.VMEM((tpw, D_MODEL), jnp.float32),
            pltpu.SemaphoreType.DMA,
        ],
    )
    def dispatch(x_hbm, idx0_hbm, idx1_hbm, out_hbm,
                 idx0_v, idx1_v, rows_v, sem):
        wid = lax.axis_index("s") * NC + lax.axis_index("c")
        base = wid * tpw
        c0 = pltpu.async_copy(idx0_hbm.at[pl.ds(base, tpw)], idx0_v, sem)
        c1 = pltpu.async_copy(idx1_hbm.at[pl.ds(base, tpw)], idx1_v, sem)
        c2 = pltpu.async_copy(x_hbm.at[pl.ds(base, tpw)], rows_v, sem)
        c0.wait()
        c1.wait()
        c2.wait()
        s0 = pltpu.async_copy(rows_v, out_hbm.at[idx0_v], sem)
        s1 = pltpu.async_copy(rows_v, out_hbm.at[idx1_v], sem)
        s0.wait()
        s1.wait()

    return dispatch


def _make_combine(n_tok):
    sub = 32
    n_sub = n_tok // (NW * sub)
    mesh = plsc.VectorSubcoreMesh(core_axis_name="c", subcore_axis_name="s",
                                  num_cores=NC, num_subcores=NS)

    @functools.partial(
        pl.kernel,
        out_type=jax.ShapeDtypeStruct((n_tok, D_MODEL), jnp.float32),
        mesh=mesh,
        scratch_types=[
            pltpu.VMEM((2, sub), jnp.int32),
            pltpu.VMEM((2, sub), jnp.int32),
            pltpu.VMEM((2, sub, 16), jnp.float32),
            pltpu.VMEM((2, sub, 16), jnp.float32),
            pltpu.VMEM((2, sub, D_MODEL), jnp.float32),
            pltpu.VMEM((2, sub, D_MODEL), jnp.float32),
            pltpu.SemaphoreType.DMA,
            pltpu.SemaphoreType.DMA,
            pltpu.SemaphoreType.DMA,
        ],
    )
    def combine(y_hbm, r0_hbm, r1_hbm, w0_hbm, w1_hbm, out_hbm,
                r0_v, r1_v, w0_v, w1_v, a_v, b_v, gsem0, gsem1, osem):
        wid = lax.axis_index("s") * NC + lax.axis_index("c")
        gsems = (gsem0, gsem1)

        def fire(s):
            k = s % 2
            base = (wid * n_sub + s) * sub
            sem = gsems[k]
            c0 = pltpu.async_copy(r0_hbm.at[pl.ds(base, sub)], r0_v.at[k],
                                  sem)
            c1 = pltpu.async_copy(r1_hbm.at[pl.ds(base, sub)], r1_v.at[k],
                                  sem)
            c2 = pltpu.async_copy(w0_hbm.at[pl.ds(base, sub)], w0_v.at[k],
                                  sem)
            c3 = pltpu.async_copy(w1_hbm.at[pl.ds(base, sub)], w1_v.at[k],
                                  sem)
            c0.wait()
            c1.wait()
            c2.wait()
            c3.wait()
            return (pltpu.async_copy(y_hbm.at[r0_v.at[k]], a_v.at[k], sem),
                    pltpu.async_copy(y_hbm.at[r1_v.at[k]], b_v.at[k], sem))

        pending = fire(0)
        prev_st = None
        for s in range(n_sub):
            k = s % 2
            pending[0].wait()
            pending[1].wait()
            if prev_st is not None:
                prev_st.wait()
            if s + 1 < n_sub:
                pending = fire(s + 1)

            def body(i, _):
                wa = w0_v[k, i, :]
                wb = w1_v[k, i, :]
                for c in range(D_MODEL // 16):
                    a_v[k, i, pl.ds(c * 16, 16)] = (
                        wa * a_v[k, i, pl.ds(c * 16, 16)]
                        + wb * b_v[k, i, pl.ds(c * 16, 16)])
                return 0

            lax.fori_loop(0, sub, body, 0)
            base = (wid * n_sub + s) * sub
            prev_st = pltpu.async_copy(a_v.at[k],
                                       out_hbm.at[pl.ds(base, sub)], osem)
        prev_st.wait()

    return combine


@jax.jit
def kernel(x, Wg1, bg1, Wg2, bg2, We1, be1, We2, be2):
    B, S, D = x.shape
    T = B * S
    x_flat = x.reshape(T, D)
    n_t = T // T_TILE

    i1, i2, w1, w2, usage_sum, rank1, rank2, counts_out = pl.pallas_call(
        _gate_kernel,
        grid=(n_t,),
        in_specs=[
            pl.BlockSpec((T_TILE, D_MODEL), lambda t: (t, 0)),
            pl.BlockSpec((D_MODEL, D_GATE_HID), lambda t: (0, 0)),
            pl.BlockSpec((1, D_GATE_HID), lambda t: (0, 0)),
            pl.BlockSpec((D_GATE_HID, NUM_EXPERTS), lambda t: (0, 0)),
            pl.BlockSpec((1, NUM_EXPERTS), lambda t: (0, 0)),
        ],
        out_specs=(
            pl.BlockSpec((T_TILE, 1), lambda t: (t, 0)),
            pl.BlockSpec((T_TILE, 1), lambda t: (t, 0)),
            pl.BlockSpec((T_TILE, 1), lambda t: (t, 0)),
            pl.BlockSpec((T_TILE, 1), lambda t: (t, 0)),
            pl.BlockSpec((1, NUM_EXPERTS), lambda t: (0, 0)),
            pl.BlockSpec((T_TILE, 1), lambda t: (t, 0)),
            pl.BlockSpec((T_TILE, 1), lambda t: (t, 0)),
            pl.BlockSpec((1, NUM_EXPERTS), lambda t: (0, 0)),
        ),
        out_shape=(
            jax.ShapeDtypeStruct((T, 1), jnp.int32),
            jax.ShapeDtypeStruct((T, 1), jnp.int32),
            jax.ShapeDtypeStruct((T, 1), jnp.float32),
            jax.ShapeDtypeStruct((T, 1), jnp.float32),
            jax.ShapeDtypeStruct((1, NUM_EXPERTS), jnp.float32),
            jax.ShapeDtypeStruct((T, 1), jnp.int32),
            jax.ShapeDtypeStruct((T, 1), jnp.int32),
            jax.ShapeDtypeStruct((1, NUM_EXPERTS), jnp.int32),
        ),
    )(x_flat, Wg1.astype(jnp.bfloat16), bg1.reshape(1, -1),
      Wg2.astype(jnp.bfloat16), bg2.reshape(1, -1))

    counts = counts_out[0]
    psize = ((counts + R_TILE - 1) // R_TILE) * R_TILE
    pstart = jnp.concatenate([jnp.zeros((1,), jnp.int32),
                              jnp.cumsum(psize)[:-1].astype(jnp.int32)])
    idx0 = pstart[i1[:, 0]] + rank1[:, 0]
    idx1 = pstart[i2[:, 0]] + rank2[:, 0]

    P = T * TOP_K + NUM_EXPERTS * R_TILE
    n_tiles = P // R_TILE
    t_starts = jnp.arange(n_tiles, dtype=jnp.int32) * R_TILE
    e_of_t = (jnp.sum((pstart[None, :] <= t_starts[:, None]), axis=1) - 1
              ).astype(jnp.int32)

    x_sorted = _make_dispatch(T, P)(x_flat, idx0, idx1)

    n_active = (pstart[-1] + psize[-1] + R_TILE - 1) // R_TILE
    y_sorted = pl.pallas_call(
        _gmm_kernel,
        grid_spec=pltpu.PrefetchScalarGridSpec(
            num_scalar_prefetch=2,
            grid=(n_tiles,),
            in_specs=[
                pl.BlockSpec((R_TILE, D_MODEL), lambda t, em, na: (t, 0)),
                pl.BlockSpec((1, D_MODEL, D_FF),
                             lambda t, em, na: (em[t], 0, 0)),
                pl.BlockSpec((1, 1, D_FF), lambda t, em, na: (em[t], 0, 0)),
                pl.BlockSpec((1, D_FF, D_MODEL),
                             lambda t, em, na: (em[t], 0, 0)),
                pl.BlockSpec((1, 1, D_MODEL),
                             lambda t, em, na: (em[t], 0, 0)),
            ],
            out_specs=pl.BlockSpec((R_TILE, D_MODEL),
                                   lambda t, em, na: (t, 0)),
        ),
        out_shape=jax.ShapeDtypeStruct((P, D_MODEL), jnp.float32),
    )(e_of_t, n_active.reshape(1), x_sorted, We1,
      be1.reshape(NUM_EXPERTS, 1, D_FF),
      We2, be2.reshape(NUM_EXPERTS, 1, D_MODEL))

    w0b = jnp.broadcast_to(w1, (T, 16))
    w1b = jnp.broadcast_to(w2, (T, 16))
    out = _make_combine(T)(y_sorted, idx0, idx1, w0b, w1b)

    usage = usage_sum[0] / T
    ideal = 1.0 / NUM_EXPERTS
    lb_loss = LB_COEF * jnp.mean((usage - ideal) ** 2)
    return out.reshape(B, S, D), lb_loss


# speedup vs baseline: 1.4175x; 1.0468x over previous
import functools

import jax
import jax.numpy as jnp
from jax import lax
from jax.experimental import pallas as pl
from jax.experimental.pallas import tpu as pltpu
from jax.experimental.pallas import tpu_sc as plsc

NUM_EXPERTS = 8
TOP_K = 2
D_MODEL = 768
D_GATE_HID = 2 * D_MODEL
D_FF = 4 * D_MODEL
LB_COEF = 0.01

T_TILE = 512
R_TILE = 256
FF_TILE = 512
N_FF = D_FF // FF_TILE

NC, NS = 2, 16
NW = NC * NS


def _gate_kernel(x_ref, wg1_ref, bg1_ref, wg2_ref, bg2_ref,
                 i1_ref, i2_ref, w1_ref, w2_ref, usage_ref,
                 rank1_ref, rank2_ref, counts_ref):
    x = x_ref[...].astype(jnp.bfloat16)
    h = jnp.maximum(jnp.dot(x, wg1_ref[...],
                            preferred_element_type=jnp.float32)
                    + bg1_ref[...], 0.0)
    logits = jnp.dot(h.astype(jnp.bfloat16), wg2_ref[...],
                     preferred_element_type=jnp.float32) + bg2_ref[...]
    m = jnp.max(logits, axis=-1, keepdims=True)
    e = jnp.exp(logits - m)
    scores = e / jnp.sum(e, axis=-1, keepdims=True)

    lane = jax.lax.broadcasted_iota(jnp.int32, scores.shape, 1)
    big = jnp.int32(NUM_EXPERTS)
    m1 = jnp.max(scores, axis=-1, keepdims=True)
    i1 = jnp.min(jnp.where(scores == m1, lane, big), axis=-1, keepdims=True)
    masked = jnp.where(lane == i1, -jnp.inf, scores)
    m2 = jnp.max(masked, axis=-1, keepdims=True)
    i2 = jnp.min(jnp.where(masked == m2, lane, big), axis=-1, keepdims=True)

    i1_ref[...] = i1
    i2_ref[...] = i2
    s = m1 + m2
    w1_ref[...] = jnp.broadcast_to(m1 / s, w1_ref.shape)
    w2_ref[...] = jnp.broadcast_to(m2 / s, w2_ref.shape)

    @pl.when(pl.program_id(0) == 0)
    def _init():
        usage_ref[...] = jnp.zeros_like(usage_ref)
        counts_ref[...] = jnp.zeros_like(counts_ref)

    usage_ref[...] += jnp.sum(scores, axis=0, keepdims=True)

    oh1 = (lane == i1).astype(jnp.float32)
    oh2 = (lane == i2).astype(jnp.float32)
    ohsum = oh1 + oh2
    r_iota = jax.lax.broadcasted_iota(jnp.int32, (T_TILE, T_TILE), 0)
    c_iota = jax.lax.broadcasted_iota(jnp.int32, (T_TILE, T_TILE), 1)
    tri = (r_iota >= c_iota).astype(jnp.float32)
    cum = jnp.dot(tri, ohsum, preferred_element_type=jnp.float32)
    carry = counts_ref[...].astype(jnp.float32) + cum - ohsum
    rank1_ref[...] = jnp.sum(carry * oh1, axis=1,
                             keepdims=True).astype(jnp.int32)
    rank2_ref[...] = jnp.sum(carry * oh2, axis=1,
                             keepdims=True).astype(jnp.int32)
    counts_ref[...] += jnp.sum(ohsum, axis=0,
                               keepdims=True).astype(jnp.int32)


def _gmm_kernel(e_map_ref, na_ref, x_ref, we1_ref, be1_ref, we2_ref,
                be2_ref, out_ref):
    @pl.when(pl.program_id(0) < na_ref[0])
    def _active():
        h = jnp.dot(x_ref[...], we1_ref[0],
                    preferred_element_type=jnp.float32,
                    precision=jax.lax.Precision.DEFAULT) + be1_ref[0]
        h = 0.5 * h * (1.0 + jax.lax.erf(h * 0.7071067811865476))
        out_ref[...] = jnp.dot(h, we2_ref[0],
                               preferred_element_type=jnp.float32,
                               precision=jax.lax.Precision.DEFAULT) + be2_ref[0]


def _make_dispatch(n_tok, p_rows):
    tpw = n_tok // NW
    mesh = plsc.VectorSubcoreMesh(core_axis_name="c", subcore_axis_name="s",
                                  num_cores=NC, num_subcores=NS)

    @functools.partial(
        pl.kernel,
        out_type=jax.ShapeDtypeStruct((p_rows, D_MODEL), jnp.float32),
        mesh=mesh,
        scratch_types=[
            pltpu.VMEM((tpw,), jnp.int32),
            pltpu.VMEM((tpw,), jnp.int32),
            pltpu.VMEM((tpw, D_MODEL), jnp.float32),
            pltpu.SemaphoreType.DMA,
        ],
    )
    def dispatch(x_hbm, idx0_hbm, idx1_hbm, out_hbm,
                 idx0_v, idx1_v, rows_v, sem):
        wid = lax.axis_index("s") * NC + lax.axis_index("c")
        base = wid * tpw
        c0 = pltpu.async_copy(idx0_hbm.at[pl.ds(base, tpw)], idx0_v, sem)
        c1 = pltpu.async_copy(idx1_hbm.at[pl.ds(base, tpw)], idx1_v, sem)
        c2 = pltpu.async_copy(x_hbm.at[pl.ds(base, tpw)], rows_v, sem)
        c0.wait()
        c1.wait()
        c2.wait()
        s0 = pltpu.async_copy(rows_v, out_hbm.at[idx0_v], sem)
        s1 = pltpu.async_copy(rows_v, out_hbm.at[idx1_v], sem)
        s0.wait()
        s1.wait()

    return dispatch


def _make_combine(n_tok):
    sub = 32
    n_sub = n_tok // (NW * sub)
    mesh = plsc.VectorSubcoreMesh(core_axis_name="c", subcore_axis_name="s",
                                  num_cores=NC, num_subcores=NS)

    @functools.partial(
        pl.kernel,
        out_type=jax.ShapeDtypeStruct((n_tok, D_MODEL), jnp.float32),
        mesh=mesh,
        scratch_types=[
            pltpu.VMEM((2, sub), jnp.int32),
            pltpu.VMEM((2, sub), jnp.int32),
            pltpu.VMEM((2, sub, 16), jnp.float32),
            pltpu.VMEM((2, sub, 16), jnp.float32),
            pltpu.VMEM((2, sub, D_MODEL), jnp.float32),
            pltpu.VMEM((2, sub, D_MODEL), jnp.float32),
            pltpu.SemaphoreType.DMA,
            pltpu.SemaphoreType.DMA,
            pltpu.SemaphoreType.DMA,
        ],
    )
    def combine(y_hbm, r0_hbm, r1_hbm, w0_hbm, w1_hbm, out_hbm,
                r0_v, r1_v, w0_v, w1_v, a_v, b_v, gsem0, gsem1, osem):
        wid = lax.axis_index("s") * NC + lax.axis_index("c")
        gsems = (gsem0, gsem1)

        def fire(s):
            k = s % 2
            base = (wid * n_sub + s) * sub
            sem = gsems[k]
            c0 = pltpu.async_copy(r0_hbm.at[pl.ds(base, sub)], r0_v.at[k],
                                  sem)
            c1 = pltpu.async_copy(r1_hbm.at[pl.ds(base, sub)], r1_v.at[k],
                                  sem)
            c2 = pltpu.async_copy(w0_hbm.at[pl.ds(base, sub)], w0_v.at[k],
                                  sem)
            c3 = pltpu.async_copy(w1_hbm.at[pl.ds(base, sub)], w1_v.at[k],
                                  sem)
            c0.wait()
            c1.wait()
            c2.wait()
            c3.wait()
            return (pltpu.async_copy(y_hbm.at[r0_v.at[k]], a_v.at[k], sem),
                    pltpu.async_copy(y_hbm.at[r1_v.at[k]], b_v.at[k], sem))

        pending = fire(0)
        prev_st = None
        for s in range(n_sub):
            k = s % 2
            pending[0].wait()
            pending[1].wait()
            if prev_st is not None:
                prev_st.wait()
            if s + 1 < n_sub:
                pending = fire(s + 1)

            def body(i, _):
                wa = w0_v[k, i, :]
                wb = w1_v[k, i, :]
                for c in range(D_MODEL // 16):
                    a_v[k, i, pl.ds(c * 16, 16)] = (
                        wa * a_v[k, i, pl.ds(c * 16, 16)]
                        + wb * b_v[k, i, pl.ds(c * 16, 16)])
                return 0

            lax.fori_loop(0, sub, body, 0)
            base = (wid * n_sub + s) * sub
            prev_st = pltpu.async_copy(a_v.at[k],
                                       out_hbm.at[pl.ds(base, sub)], osem)
        prev_st.wait()

    return combine


@jax.jit
def kernel(x, Wg1, bg1, Wg2, bg2, We1, be1, We2, be2):
    B, S, D = x.shape
    T = B * S
    x_flat = x.reshape(T, D)
    n_t = T // T_TILE

    i1, i2, w1, w2, usage_sum, rank1, rank2, counts_out = pl.pallas_call(
        _gate_kernel,
        grid=(n_t,),
        in_specs=[
            pl.BlockSpec((T_TILE, D_MODEL), lambda t: (t, 0)),
            pl.BlockSpec((D_MODEL, D_GATE_HID), lambda t: (0, 0)),
            pl.BlockSpec((1, D_GATE_HID), lambda t: (0, 0)),
            pl.BlockSpec((D_GATE_HID, NUM_EXPERTS), lambda t: (0, 0)),
            pl.BlockSpec((1, NUM_EXPERTS), lambda t: (0, 0)),
        ],
        out_specs=(
            pl.BlockSpec((T_TILE, 1), lambda t: (t, 0)),
            pl.BlockSpec((T_TILE, 1), lambda t: (t, 0)),
            pl.BlockSpec((T_TILE, 16), lambda t: (t, 0)),
            pl.BlockSpec((T_TILE, 16), lambda t: (t, 0)),
            pl.BlockSpec((1, NUM_EXPERTS), lambda t: (0, 0)),
            pl.BlockSpec((T_TILE, 1), lambda t: (t, 0)),
            pl.BlockSpec((T_TILE, 1), lambda t: (t, 0)),
            pl.BlockSpec((1, NUM_EXPERTS), lambda t: (0, 0)),
        ),
        out_shape=(
            jax.ShapeDtypeStruct((T, 1), jnp.int32),
            jax.ShapeDtypeStruct((T, 1), jnp.int32),
            jax.ShapeDtypeStruct((T, 16), jnp.float32),
            jax.ShapeDtypeStruct((T, 16), jnp.float32),
            jax.ShapeDtypeStruct((1, NUM_EXPERTS), jnp.float32),
            jax.ShapeDtypeStruct((T, 1), jnp.int32),
            jax.ShapeDtypeStruct((T, 1), jnp.int32),
            jax.ShapeDtypeStruct((1, NUM_EXPERTS), jnp.int32),
        ),
    )(x_flat, Wg1.astype(jnp.bfloat16), bg1.reshape(1, -1),
      Wg2.astype(jnp.bfloat16), bg2.reshape(1, -1))

    counts = counts_out[0]
    psize = ((counts + R_TILE - 1) // R_TILE) * R_TILE
    pstart = jnp.concatenate([jnp.zeros((1,), jnp.int32),
                              jnp.cumsum(psize)[:-1].astype(jnp.int32)])
    idx0 = pstart[i1[:, 0]] + rank1[:, 0]
    idx1 = pstart[i2[:, 0]] + rank2[:, 0]

    P = T * TOP_K + NUM_EXPERTS * R_TILE
    n_tiles = P // R_TILE
    t_starts = jnp.arange(n_tiles, dtype=jnp.int32) * R_TILE
    e_of_t = (jnp.sum((pstart[None, :] <= t_starts[:, None]), axis=1) - 1
              ).astype(jnp.int32)

    x_sorted = _make_dispatch(T, P)(x_flat, idx0, idx1)

    n_active = (pstart[-1] + psize[-1] + R_TILE - 1) // R_TILE
    y_sorted = pl.pallas_call(
        _gmm_kernel,
        grid_spec=pltpu.PrefetchScalarGridSpec(
            num_scalar_prefetch=2,
            grid=(n_tiles,),
            in_specs=[
                pl.BlockSpec((R_TILE, D_MODEL), lambda t, em, na: (t, 0)),
                pl.BlockSpec((1, D_MODEL, D_FF),
                             lambda t, em, na: (em[t], 0, 0)),
                pl.BlockSpec((1, 1, D_FF), lambda t, em, na: (em[t], 0, 0)),
                pl.BlockSpec((1, D_FF, D_MODEL),
                             lambda t, em, na: (em[t], 0, 0)),
                pl.BlockSpec((1, 1, D_MODEL),
                             lambda t, em, na: (em[t], 0, 0)),
            ],
            out_specs=pl.BlockSpec((R_TILE, D_MODEL),
                                   lambda t, em, na: (t, 0)),
        ),
        out_shape=jax.ShapeDtypeStruct((P, D_MODEL), jnp.float32),
    )(e_of_t, n_active.reshape(1), x_sorted, We1,
      be1.reshape(NUM_EXPERTS, 1, D_FF),
      We2, be2.reshape(NUM_EXPERTS, 1, D_MODEL))

    out = _make_combine(T)(y_sorted, idx0, idx1, w1, w2)

    usage = usage_sum[0] / T
    ideal = 1.0 / NUM_EXPERTS
    lb_loss = LB_COEF * jnp.mean((usage - ideal) ** 2)
    return out.reshape(B, S, D), lb_loss


# speedup vs baseline: 1.4197x; 1.0015x over previous
import functools

import jax
import jax.numpy as jnp
from jax import lax
from jax.experimental import pallas as pl
from jax.experimental.pallas import tpu as pltpu
from jax.experimental.pallas import tpu_sc as plsc

NUM_EXPERTS = 8
TOP_K = 2
D_MODEL = 768
D_GATE_HID = 2 * D_MODEL
D_FF = 4 * D_MODEL
LB_COEF = 0.01

T_TILE = 1024
R_TILE = 256
FF_TILE = 512
N_FF = D_FF // FF_TILE

NC, NS = 2, 16
NW = NC * NS


def _gate_kernel(x_ref, wg1_ref, bg1_ref, wg2_ref, bg2_ref,
                 i1_ref, i2_ref, w1_ref, w2_ref, usage_ref,
                 rank1_ref, rank2_ref, counts_ref):
    x = x_ref[...].astype(jnp.bfloat16)
    h = jnp.maximum(jnp.dot(x, wg1_ref[...],
                            preferred_element_type=jnp.float32)
                    + bg1_ref[...], 0.0)
    logits = jnp.dot(h.astype(jnp.bfloat16), wg2_ref[...],
                     preferred_element_type=jnp.float32) + bg2_ref[...]
    m = jnp.max(logits, axis=-1, keepdims=True)
    e = jnp.exp(logits - m)
    scores = e / jnp.sum(e, axis=-1, keepdims=True)

    lane = jax.lax.broadcasted_iota(jnp.int32, scores.shape, 1)
    big = jnp.int32(NUM_EXPERTS)
    m1 = jnp.max(scores, axis=-1, keepdims=True)
    i1 = jnp.min(jnp.where(scores == m1, lane, big), axis=-1, keepdims=True)
    masked = jnp.where(lane == i1, -jnp.inf, scores)
    m2 = jnp.max(masked, axis=-1, keepdims=True)
    i2 = jnp.min(jnp.where(masked == m2, lane, big), axis=-1, keepdims=True)

    i1_ref[...] = i1
    i2_ref[...] = i2
    s = m1 + m2
    w1_ref[...] = jnp.broadcast_to(m1 / s, w1_ref.shape)
    w2_ref[...] = jnp.broadcast_to(m2 / s, w2_ref.shape)

    @pl.when(pl.program_id(0) == 0)
    def _init():
        usage_ref[...] = jnp.zeros_like(usage_ref)
        counts_ref[...] = jnp.zeros_like(counts_ref)

    usage_ref[...] += jnp.sum(scores, axis=0, keepdims=True)

    oh1 = (lane == i1).astype(jnp.float32)
    oh2 = (lane == i2).astype(jnp.float32)
    ohsum = oh1 + oh2
    r_iota = jax.lax.broadcasted_iota(jnp.int32, (T_TILE, T_TILE), 0)
    c_iota = jax.lax.broadcasted_iota(jnp.int32, (T_TILE, T_TILE), 1)
    tri = (r_iota >= c_iota).astype(jnp.float32)
    cum = jnp.dot(tri, ohsum, preferred_element_type=jnp.float32)
    carry = counts_ref[...].astype(jnp.float32) + cum - ohsum
    rank1_ref[...] = jnp.sum(carry * oh1, axis=1,
                             keepdims=True).astype(jnp.int32)
    rank2_ref[...] = jnp.sum(carry * oh2, axis=1,
                             keepdims=True).astype(jnp.int32)
    counts_ref[...] += jnp.sum(ohsum, axis=0,
                               keepdims=True).astype(jnp.int32)


def _gmm_kernel(e_map_ref, na_ref, x_ref, we1_ref, be1_ref, we2_ref,
                be2_ref, out_ref):
    @pl.when(pl.program_id(0) < na_ref[0])
    def _active():
        h = jnp.dot(x_ref[...], we1_ref[0],
                    preferred_element_type=jnp.float32,
                    precision=jax.lax.Precision.DEFAULT) + be1_ref[0]
        h = 0.5 * h * (1.0 + jax.lax.erf(h * 0.7071067811865476))
        out_ref[...] = jnp.dot(h, we2_ref[0],
                               preferred_element_type=jnp.float32,
                               precision=jax.lax.Precision.DEFAULT) + be2_ref[0]


def _make_dispatch(n_tok, p_rows):
    tpw = n_tok // NW
    mesh = plsc.VectorSubcoreMesh(core_axis_name="c", subcore_axis_name="s",
                                  num_cores=NC, num_subcores=NS)

    @functools.partial(
        pl.kernel,
        out_type=jax.ShapeDtypeStruct((p_rows, D_MODEL), jnp.float32),
        mesh=mesh,
        scratch_types=[
            pltpu.VMEM((tpw,), jnp.int32),
            pltpu.VMEM((tpw,), jnp.int32),
            pltpu.VMEM((tpw, D_MODEL), jnp.float32),
            pltpu.SemaphoreType.DMA,
        ],
    )
    def dispatch(x_hbm, idx0_hbm, idx1_hbm, out_hbm,
                 idx0_v, idx1_v, rows_v, sem):
        wid = lax.axis_index("s") * NC + lax.axis_index("c")
        base = wid * tpw
        c0 = pltpu.async_copy(idx0_hbm.at[pl.ds(base, tpw)], idx0_v, sem)
        c1 = pltpu.async_copy(idx1_hbm.at[pl.ds(base, tpw)], idx1_v, sem)
        c2 = pltpu.async_copy(x_hbm.at[pl.ds(base, tpw)], rows_v, sem)
        c0.wait()
        c1.wait()
        c2.wait()
        s0 = pltpu.async_copy(rows_v, out_hbm.at[idx0_v], sem)
        s1 = pltpu.async_copy(rows_v, out_hbm.at[idx1_v], sem)
        s0.wait()
        s1.wait()

    return dispatch


def _make_combine(n_tok):
    sub = 32
    n_sub = n_tok // (NW * sub)
    mesh = plsc.VectorSubcoreMesh(core_axis_name="c", subcore_axis_name="s",
                                  num_cores=NC, num_subcores=NS)

    @functools.partial(
        pl.kernel,
        out_type=jax.ShapeDtypeStruct((n_tok, D_MODEL), jnp.float32),
        mesh=mesh,
        scratch_types=[
            pltpu.VMEM((2, sub), jnp.int32),
            pltpu.VMEM((2, sub), jnp.int32),
            pltpu.VMEM((2, sub, 16), jnp.float32),
            pltpu.VMEM((2, sub, 16), jnp.float32),
            pltpu.VMEM((2, sub, D_MODEL), jnp.float32),
            pltpu.VMEM((2, sub, D_MODEL), jnp.float32),
            pltpu.SemaphoreType.DMA,
            pltpu.SemaphoreType.DMA,
            pltpu.SemaphoreType.DMA,
        ],
    )
    def combine(y_hbm, r0_hbm, r1_hbm, w0_hbm, w1_hbm, out_hbm,
                r0_v, r1_v, w0_v, w1_v, a_v, b_v, gsem0, gsem1, osem):
        wid = lax.axis_index("s") * NC + lax.axis_index("c")
        gsems = (gsem0, gsem1)

        def fire(s):
            k = s % 2
            base = (wid * n_sub + s) * sub
            sem = gsems[k]
            c0 = pltpu.async_copy(r0_hbm.at[pl.ds(base, sub)], r0_v.at[k],
                                  sem)
            c1 = pltpu.async_copy(r1_hbm.at[pl.ds(base, sub)], r1_v.at[k],
                                  sem)
            c2 = pltpu.async_copy(w0_hbm.at[pl.ds(base, sub)], w0_v.at[k],
                                  sem)
            c3 = pltpu.async_copy(w1_hbm.at[pl.ds(base, sub)], w1_v.at[k],
                                  sem)
            c0.wait()
            c1.wait()
            c2.wait()
            c3.wait()
            return (pltpu.async_copy(y_hbm.at[r0_v.at[k]], a_v.at[k], sem),
                    pltpu.async_copy(y_hbm.at[r1_v.at[k]], b_v.at[k], sem))

        pending = fire(0)
        prev_st = None
        for s in range(n_sub):
            k = s % 2
            pending[0].wait()
            pending[1].wait()
            if prev_st is not None:
                prev_st.wait()
            if s + 1 < n_sub:
                pending = fire(s + 1)

            def body(i, _):
                wa = w0_v[k, i, :]
                wb = w1_v[k, i, :]
                for c in range(D_MODEL // 16):
                    a_v[k, i, pl.ds(c * 16, 16)] = (
                        wa * a_v[k, i, pl.ds(c * 16, 16)]
                        + wb * b_v[k, i, pl.ds(c * 16, 16)])
                return 0

            lax.fori_loop(0, sub, body, 0)
            base = (wid * n_sub + s) * sub
            prev_st = pltpu.async_copy(a_v.at[k],
                                       out_hbm.at[pl.ds(base, sub)], osem)
        prev_st.wait()

    return combine


@jax.jit
def kernel(x, Wg1, bg1, Wg2, bg2, We1, be1, We2, be2):
    B, S, D = x.shape
    T = B * S
    x_flat = x.reshape(T, D)
    n_t = T // T_TILE

    i1, i2, w1, w2, usage_sum, rank1, rank2, counts_out = pl.pallas_call(
        _gate_kernel,
        grid=(n_t,),
        in_specs=[
            pl.BlockSpec((T_TILE, D_MODEL), lambda t: (t, 0)),
            pl.BlockSpec((D_MODEL, D_GATE_HID), lambda t: (0, 0)),
            pl.BlockSpec((1, D_GATE_HID), lambda t: (0, 0)),
            pl.BlockSpec((D_GATE_HID, NUM_EXPERTS), lambda t: (0, 0)),
            pl.BlockSpec((1, NUM_EXPERTS), lambda t: (0, 0)),
        ],
        out_specs=(
            pl.BlockSpec((T_TILE, 1), lambda t: (t, 0)),
            pl.BlockSpec((T_TILE, 1), lambda t: (t, 0)),
            pl.BlockSpec((T_TILE, 16), lambda t: (t, 0)),
            pl.BlockSpec((T_TILE, 16), lambda t: (t, 0)),
            pl.BlockSpec((1, NUM_EXPERTS), lambda t: (0, 0)),
            pl.BlockSpec((T_TILE, 1), lambda t: (t, 0)),
            pl.BlockSpec((T_TILE, 1), lambda t: (t, 0)),
            pl.BlockSpec((1, NUM_EXPERTS), lambda t: (0, 0)),
        ),
        out_shape=(
            jax.ShapeDtypeStruct((T, 1), jnp.int32),
            jax.ShapeDtypeStruct((T, 1), jnp.int32),
            jax.ShapeDtypeStruct((T, 16), jnp.float32),
            jax.ShapeDtypeStruct((T, 16), jnp.float32),
            jax.ShapeDtypeStruct((1, NUM_EXPERTS), jnp.float32),
            jax.ShapeDtypeStruct((T, 1), jnp.int32),
            jax.ShapeDtypeStruct((T, 1), jnp.int32),
            jax.ShapeDtypeStruct((1, NUM_EXPERTS), jnp.int32),
        ),
    )(x_flat, Wg1.astype(jnp.bfloat16), bg1.reshape(1, -1),
      Wg2.astype(jnp.bfloat16), bg2.reshape(1, -1))

    counts = counts_out[0]
    psize = ((counts + R_TILE - 1) // R_TILE) * R_TILE
    pstart = jnp.concatenate([jnp.zeros((1,), jnp.int32),
                              jnp.cumsum(psize)[:-1].astype(jnp.int32)])
    idx0 = pstart[i1[:, 0]] + rank1[:, 0]
    idx1 = pstart[i2[:, 0]] + rank2[:, 0]

    P = T * TOP_K + NUM_EXPERTS * R_TILE
    n_tiles = P // R_TILE
    t_starts = jnp.arange(n_tiles, dtype=jnp.int32) * R_TILE
    e_of_t = (jnp.sum((pstart[None, :] <= t_starts[:, None]), axis=1) - 1
              ).astype(jnp.int32)

    x_sorted = _make_dispatch(T, P)(x_flat, idx0, idx1)

    n_active = (pstart[-1] + psize[-1] + R_TILE - 1) // R_TILE
    y_sorted = pl.pallas_call(
        _gmm_kernel,
        grid_spec=pltpu.PrefetchScalarGridSpec(
            num_scalar_prefetch=2,
            grid=(n_tiles,),
            in_specs=[
                pl.BlockSpec((R_TILE, D_MODEL), lambda t, em, na: (t, 0)),
                pl.BlockSpec((1, D_MODEL, D_FF),
                             lambda t, em, na: (em[t], 0, 0)),
                pl.BlockSpec((1, 1, D_FF), lambda t, em, na: (em[t], 0, 0)),
                pl.BlockSpec((1, D_FF, D_MODEL),
                             lambda t, em, na: (em[t], 0, 0)),
                pl.BlockSpec((1, 1, D_MODEL),
                             lambda t, em, na: (em[t], 0, 0)),
            ],
            out_specs=pl.BlockSpec((R_TILE, D_MODEL),
                                   lambda t, em, na: (t, 0)),
        ),
        out_shape=jax.ShapeDtypeStruct((P, D_MODEL), jnp.float32),
    )(e_of_t, n_active.reshape(1), x_sorted, We1,
      be1.reshape(NUM_EXPERTS, 1, D_FF),
      We2, be2.reshape(NUM_EXPERTS, 1, D_MODEL))

    out = _make_combine(T)(y_sorted, idx0, idx1, w1, w2)

    usage = usage_sum[0] / T
    ideal = 1.0 / NUM_EXPERTS
    lb_loss = LB_COEF * jnp.mean((usage - ideal) ** 2)
    return out.reshape(B, S, D), lb_loss
